# baseline (speedup 1.0000x reference)
import functools

import jax
import jax.numpy as jnp
from jax import lax
from jax.experimental import pallas as pl
from jax.experimental.pallas import tpu as pltpu
from jax.experimental.pallas import tpu_sc as plsc

_NUM_CORES = 2
_NUM_SUBCORES = 16
_NW = _NUM_CORES * _NUM_SUBCORES

_NB = 4
_LANES = 128
_WCH = 125


def _make_widen(V: int, D: int):
    rows_per_w = V // _NW
    n_chunks = rows_per_w // _WCH
    assert n_chunks % 2 == 0
    n_pairs = n_chunks // 2
    mesh = plsc.VectorSubcoreMesh(core_axis_name="c", subcore_axis_name="s")

    @functools.partial(
        pl.kernel,
        out_type=jax.ShapeDtypeStruct((V, _LANES), jnp.float32),
        mesh=mesh,
        scratch_types=[
            pltpu.VMEM((_WCH, D), jnp.float32),
            pltpu.VMEM((_WCH, D), jnp.float32),
            pltpu.SemaphoreType.DMA,
            pltpu.SemaphoreType.DMA,
            pltpu.SemaphoreType.DMA,
            pltpu.SemaphoreType.DMA,
        ],
    )
    def widen_kernel(emb_hbm, wide_hbm, buf0, buf1, rs0, rs1, ws0, ws1):
        wid = lax.axis_index("s") * _NUM_CORES + lax.axis_index("c")
        base = wid * rows_per_w

        def fire_read(ci, buf, rsem):
            pltpu.async_copy(
                emb_hbm.at[pl.ds(base + ci * _WCH, _WCH)], buf, rsem)

        def drain_read(buf, rsem):
            pltpu.make_async_copy(
                emb_hbm.at[pl.ds(0, _WCH)], buf, rsem).wait()

        def fire_write(ci, buf, wsem):
            pltpu.async_copy(
                buf,
                wide_hbm.at[pl.ds(base + ci * _WCH, _WCH), pl.ds(0, D)],
                wsem)

        def drain_write(buf, wsem):
            pltpu.make_async_copy(
                buf, wide_hbm.at[pl.ds(0, _WCH), pl.ds(0, D)], wsem).wait()

        fire_read(0, buf0, rs0)
        fire_read(1, buf1, rs1)

        def body(g, carry):
            ci = 2 * g
            drain_read(buf0, rs0)
            fire_write(ci, buf0, ws0)
            drain_read(buf1, rs1)
            fire_write(ci + 1, buf1, ws1)
            drain_write(buf0, ws0)
            fire_read(ci + 2, buf0, rs0)
            drain_write(buf1, ws1)
            fire_read(ci + 3, buf1, rs1)
            return carry

        lax.fori_loop(0, n_pairs - 1, body, 0)

        ci = n_chunks - 2
        drain_read(buf0, rs0)
        fire_write(ci, buf0, ws0)
        drain_read(buf1, rs1)
        fire_write(ci + 1, buf1, ws1)
        drain_write(buf0, ws0)
        drain_write(buf1, ws1)

    return widen_kernel


def _make_gather(Bq: int, H: int, V: int, D: int, Hp: int):
    rows_per_w = Bq // _NW
    n_chunks = rows_per_w // _NB
    assert n_chunks % 2 == 0 and n_chunks >= 4
    n_pairs = n_chunks // 2
    mesh = plsc.VectorSubcoreMesh(core_axis_name="c", subcore_axis_name="s")

    @functools.partial(
        pl.kernel,
        out_type=jax.ShapeDtypeStruct((Bq * Hp, _LANES), jnp.float32),
        mesh=mesh,
        scratch_types=[
            pltpu.VMEM((rows_per_w, H), jnp.int32),
            pltpu.VMEM((_NB, H, _LANES), jnp.float32),
            pltpu.VMEM((_NB, H, _LANES), jnp.float32),
            pltpu.SemaphoreType.DMA,
            pltpu.SemaphoreType.DMA,
            pltpu.SemaphoreType.DMA,
            pltpu.SemaphoreType.DMA,
        ],
        compiler_params=pltpu.CompilerParams(use_tc_tiling_on_sc=False),
    )
    def gather_kernel(seq_hbm, wide_hbm, out_hbm, idx_v, buf0, buf1,
                      gsem0, gsem1, wsem0, wsem1):
        wid = lax.axis_index("s") * _NUM_CORES + lax.axis_index("c")
        base = wid * rows_per_w
        pltpu.sync_copy(seq_hbm.at[pl.ds(base, rows_per_w)], idx_v)

        def fire(ci, buf, gsem):
            for j in range(_NB):
                pltpu.async_copy(
                    wide_hbm.at[idx_v.at[ci * _NB + j]],
                    buf.at[j],
                    gsem,
                )

        def drain_gathers(buf, gsem):
            for j in range(_NB):
                pltpu.make_async_copy(
                    wide_hbm.at[pl.ds(0, H)],
                    buf.at[j], gsem).wait()

        def start_wb(ci, buf, wsem):
            for j in range(_NB):
                bb = base + ci * _NB + j
                pltpu.async_copy(
                    buf.at[j, :, pl.ds(0, D)],
                    out_hbm.at[pl.ds(bb * Hp, H), pl.ds(0, D)],
                    wsem,
                )

        def drain_wb(buf, wsem):
            for j in range(_NB):
                pltpu.make_async_copy(
                    buf.at[j, :, pl.ds(0, D)],
                    out_hbm.at[pl.ds(0, H), pl.ds(0, D)], wsem).wait()

        fire(0, buf0, gsem0)
        fire(1, buf1, gsem1)

        def body(g, carry):
            ci = 2 * g
            drain_gathers(buf0, gsem0)
            start_wb(ci, buf0, wsem0)
            drain_gathers(buf1, gsem1)
            start_wb(ci + 1, buf1, wsem1)
            drain_wb(buf0, wsem0)
            fire(ci + 2, buf0, gsem0)
            drain_wb(buf1, wsem1)
            fire(ci + 3, buf1, gsem1)
            return carry

        lax.fori_loop(0, n_pairs - 1, body, 0)

        ci = n_chunks - 2
        drain_gathers(buf0, gsem0)
        start_wb(ci, buf0, wsem0)
        drain_gathers(buf1, gsem1)
        start_wb(ci + 1, buf1, wsem1)
        drain_wb(buf0, wsem0)
        drain_wb(buf1, wsem1)

    return gather_kernel


def kernel(sequence, embeddings):
    Bq, H = sequence.shape
    V, D = embeddings.shape
    Hp = (H + 7) // 8 * 8
    wide = jnp.pad(embeddings, ((0, 0), (0, _LANES - D)))
    out128 = _make_gather(Bq, H, V, D, Hp)(sequence.astype(jnp.int32), wide)
    out3 = out128.reshape(Bq, Hp, _LANES)
    return out3[:, :H, :D]


# speedup vs baseline: 1.2967x; 1.2967x over previous
import functools

import jax
import jax.numpy as jnp
from jax import lax
from jax.experimental import pallas as pl
from jax.experimental.pallas import tpu as pltpu
from jax.experimental.pallas import tpu_sc as plsc

_NUM_CORES = 2
_NUM_SUBCORES = 16
_NW = _NUM_CORES * _NUM_SUBCORES

_NB = 16
_LANES = 128


def _make_gather(Bq: int, H: int, V: int, D: int, Hp: int):
    rows_per_w = Bq // _NW
    n_chunks = rows_per_w // _NB
    assert n_chunks % 2 == 0 and n_chunks >= 4
    n_pairs = n_chunks // 2
    mesh = plsc.VectorSubcoreMesh(core_axis_name="c", subcore_axis_name="s")

    @functools.partial(
        pl.kernel,
        out_type=jax.ShapeDtypeStruct((Bq * Hp, _LANES), jnp.float32),
        mesh=mesh,
        scratch_types=[
            pltpu.VMEM((rows_per_w, H), jnp.int32),
            pltpu.VMEM((_NB, H, D), jnp.float32),
            pltpu.VMEM((_NB, H, D), jnp.float32),
            pltpu.SemaphoreType.DMA,
            pltpu.SemaphoreType.DMA,
            pltpu.SemaphoreType.DMA,
            pltpu.SemaphoreType.DMA,
        ],
        compiler_params=pltpu.CompilerParams(use_tc_tiling_on_sc=False),
    )
    def gather_kernel(seq_hbm, table_hbm, out_hbm, idx_v, buf0, buf1,
                      gsem0, gsem1, wsem0, wsem1):
        wid = lax.axis_index("s") * _NUM_CORES + lax.axis_index("c")
        base = wid * rows_per_w
        pltpu.sync_copy(seq_hbm.at[pl.ds(base, rows_per_w)], idx_v)

        def fire(ci, buf, gsem):
            for j in range(_NB):
                pltpu.async_copy(
                    table_hbm.at[idx_v.at[ci * _NB + j]],
                    buf.at[j],
                    gsem,
                )

        def drain_gathers(buf, gsem):
            for j in range(_NB):
                pltpu.make_async_copy(
                    table_hbm.at[idx_v.at[j]], buf.at[j], gsem).wait()

        def start_wb(ci, buf, wsem):
            for j in range(_NB):
                bb = base + ci * _NB + j
                pltpu.async_copy(
                    buf.at[j],
                    out_hbm.at[pl.ds(bb * Hp, H), pl.ds(0, D)],
                    wsem,
                )

        def drain_wb(buf, wsem):
            for j in range(_NB):
                pltpu.make_async_copy(
                    buf.at[j],
                    out_hbm.at[pl.ds(0, H), pl.ds(0, D)], wsem).wait()

        fire(0, buf0, gsem0)
        fire(1, buf1, gsem1)

        def body(g, carry):
            ci = 2 * g
            drain_gathers(buf0, gsem0)
            start_wb(ci, buf0, wsem0)
            drain_gathers(buf1, gsem1)
            start_wb(ci + 1, buf1, wsem1)
            drain_wb(buf0, wsem0)
            fire(ci + 2, buf0, gsem0)
            drain_wb(buf1, wsem1)
            fire(ci + 3, buf1, gsem1)
            return carry

        lax.fori_loop(0, n_pairs - 1, body, 0)

        ci = n_chunks - 2
        drain_gathers(buf0, gsem0)
        start_wb(ci, buf0, wsem0)
        drain_gathers(buf1, gsem1)
        start_wb(ci + 1, buf1, wsem1)
        drain_wb(buf0, wsem0)
        drain_wb(buf1, wsem1)

    return gather_kernel


def kernel(sequence, embeddings):
    Bq, H = sequence.shape
    V, D = embeddings.shape
    Hp = (H + 7) // 8 * 8
    out128 = _make_gather(Bq, H, V, D, Hp)(sequence.astype(jnp.int32),
                                           embeddings)
    out3 = out128.reshape(Bq, Hp, _LANES)
    return out3[:, :H, :D]
